# baseline (speedup 1.0000x reference)
import functools

import jax
import jax.numpy as jnp
from jax.experimental import pallas as pl


def _entmax_mask_kernel(w_ref, mask_ref):
    z = w_ref[...] * 0.5
    zmax = jnp.max(z, axis=-1, keepdims=True)
    lo = zmax - 1.0
    hi = zmax

    def body(_, carry):
        lo, hi = carry
        mid = (lo + hi) * 0.5
        t = jnp.maximum(z - mid, 0.0)
        f = jnp.sum(t * t, axis=-1, keepdims=True)
        gt = f > 1.0
        lo = jnp.where(gt, mid, lo)
        hi = jnp.where(gt, hi, mid)
        return lo, hi

    lo, hi = jax.lax.fori_loop(0, 40, body, (lo, hi))
    tau = (lo + hi) * 0.5
    p = jnp.maximum(z - tau, 0.0)
    mask_ref[...] = p * p


def _bcast_mul_kernel(mask_ref, x_ref, out_ref):
    out_ref[...] = x_ref[...][:, None, :] * mask_ref[...][None, :, :]


@jax.jit
def kernel(x, W):
    n_path, d = W.shape
    batch = x.shape[0]

    mask = pl.pallas_call(
        _entmax_mask_kernel,
        out_shape=jax.ShapeDtypeStruct((n_path, d), jnp.float32),
    )(W)

    bb = 64
    out = pl.pallas_call(
        _bcast_mul_kernel,
        grid=(batch // bb,),
        in_specs=[
            pl.BlockSpec((n_path, d), lambda i: (0, 0)),
            pl.BlockSpec((bb, d), lambda i: (i, 0)),
        ],
        out_specs=pl.BlockSpec((bb, n_path, d), lambda i: (i, 0, 0)),
        out_shape=jax.ShapeDtypeStruct((batch, n_path, d), jnp.float32),
    )(mask, x)
    return out


# speedup vs baseline: 1.0374x; 1.0374x over previous
import jax
import jax.numpy as jnp
from jax.experimental import pallas as pl
from jax.experimental.pallas import tpu as pltpu


def _fused_kernel(w_ref, x_ref, out_ref, mask_ref):
    @pl.when(pl.program_id(0) == 0)
    def _():
        z = w_ref[...] * 0.5
        zmax = jnp.max(z, axis=-1, keepdims=True)
        tau = zmax - 1.0

        def body(_, tau):
            t = jnp.maximum(z - tau, 0.0)
            f = jnp.sum(t * t, axis=-1, keepdims=True) - 1.0
            fp = -2.0 * jnp.sum(t, axis=-1, keepdims=True)
            return tau - f / fp

        tau = jax.lax.fori_loop(0, 10, body, tau)
        p = jnp.maximum(z - tau, 0.0)
        mask_ref[...] = p * p

    out_ref[...] = x_ref[...][:, None, :] * mask_ref[...][None, :, :]


@jax.jit
def kernel(x, W):
    n_path, d = W.shape
    batch = x.shape[0]
    bb = 64

    out = pl.pallas_call(
        _fused_kernel,
        grid=(batch // bb,),
        in_specs=[
            pl.BlockSpec((n_path, d), lambda i: (0, 0)),
            pl.BlockSpec((bb, d), lambda i: (i, 0)),
        ],
        out_specs=pl.BlockSpec((bb, n_path, d), lambda i: (i, 0, 0)),
        out_shape=jax.ShapeDtypeStruct((batch, n_path, d), jnp.float32),
        scratch_shapes=[pltpu.VMEM((n_path, d), jnp.float32)],
    )(W, x)
    return out


# speedup vs baseline: 1.1642x; 1.1222x over previous
import jax
import jax.numpy as jnp
from jax.experimental import pallas as pl
from jax.experimental.pallas import tpu as pltpu


def _fused_kernel(w_ref, x_ref, out_ref, mask_ref):
    @pl.when(pl.program_id(0) == 0)
    def _():
        z = w_ref[...] * 0.5
        zmax = jnp.max(z, axis=-1, keepdims=True)
        tau = zmax - 1.0

        def body(_, tau):
            t = jnp.maximum(z - tau, 0.0)
            f = jnp.sum(t * t, axis=-1, keepdims=True) - 1.0
            fp = -2.0 * jnp.sum(t, axis=-1, keepdims=True)
            return tau - f / fp

        tau = jax.lax.fori_loop(0, 10, body, tau)
        p = jnp.maximum(z - tau, 0.0)
        mask_ref[...] = p * p

    out_ref[...] = x_ref[...][:, None, :] * mask_ref[...][None, :, :]


@jax.jit
def kernel(x, W):
    n_path, d = W.shape
    batch = x.shape[0]
    bb = 128

    out = pl.pallas_call(
        _fused_kernel,
        grid=(batch // bb,),
        in_specs=[
            pl.BlockSpec((n_path, d), lambda i: (0, 0)),
            pl.BlockSpec((bb, d), lambda i: (i, 0)),
        ],
        out_specs=pl.BlockSpec((bb, n_path, d), lambda i: (i, 0, 0)),
        out_shape=jax.ShapeDtypeStruct((batch, n_path, d), jnp.float32),
        scratch_shapes=[pltpu.VMEM((n_path, d), jnp.float32)],
    )(W, x)
    return out


# speedup vs baseline: 1.1820x; 1.0154x over previous
import jax
import jax.numpy as jnp
from jax.experimental import pallas as pl
from jax.experimental.pallas import tpu as pltpu


def _fused_kernel(w_ref, x_ref, out_ref, mask_ref):
    @pl.when(pl.program_id(0) == 0)
    def _():
        z = w_ref[...] * 0.5
        zmax = jnp.max(z, axis=-1, keepdims=True)
        tau = zmax - 1.0

        def body(_, tau):
            t = jnp.maximum(z - tau, 0.0)
            f = jnp.sum(t * t, axis=-1, keepdims=True) - 1.0
            fp = -2.0 * jnp.sum(t, axis=-1, keepdims=True)
            return tau - f / fp

        tau = jax.lax.fori_loop(0, 10, body, tau)
        p = jnp.maximum(z - tau, 0.0)
        mask_ref[...] = p * p

    out_ref[...] = x_ref[...][:, None, :] * mask_ref[...][None, :, :]


@jax.jit
def kernel(x, W):
    n_path, d = W.shape
    batch = x.shape[0]
    bb = 256

    out = pl.pallas_call(
        _fused_kernel,
        grid=(batch // bb,),
        in_specs=[
            pl.BlockSpec((n_path, d), lambda i: (0, 0)),
            pl.BlockSpec((bb, d), lambda i: (i, 0)),
        ],
        out_specs=pl.BlockSpec((bb, n_path, d), lambda i: (i, 0, 0)),
        out_shape=jax.ShapeDtypeStruct((batch, n_path, d), jnp.float32),
        scratch_shapes=[pltpu.VMEM((n_path, d), jnp.float32)],
    )(W, x)
    return out
